# baseline (speedup 1.0000x reference)
import functools

import jax
import jax.numpy as jnp
from jax.experimental import pallas as pl
from jax.experimental.pallas import tpu as pltpu

_BM = 400


def _sage_kernel(adj_ref, x_ref, wl_ref, wr_ref, bl_ref, out_ref,
                 buf_ref, sem, *, bm, n_blocks):
    dot = functools.partial(
        jax.lax.dot_general,
        dimension_numbers=(((1,), (0,)), ((), ())),
        precision=jax.lax.Precision.DEFAULT,
        preferred_element_type=jnp.float32)

    def copy_in(i, slot):
        return pltpu.make_async_copy(
            adj_ref.at[pl.ds(i * bm, bm), :], buf_ref.at[slot], sem.at[slot])

    copy_in(0, 0).start()
    if n_blocks > 1:
        copy_in(1, 1).start()
    for i in range(n_blocks):
        slot = i % 2
        copy_in(i, slot).wait()
        agg = dot(buf_ref[slot], x_ref[...])
        out = dot(agg, wl_ref[...])
        out += dot(x_ref[pl.ds(i * bm, bm), :], wr_ref[...])
        out_ref[pl.ds(i * bm, bm), :] = out + bl_ref[...]
        if i + 2 < n_blocks:
            copy_in(i + 2, slot).start()


@jax.jit
def kernel(x, adj, W_l, b_l, W_r):
    n_dst, n_src = adj.shape
    d_in = x.shape[1]
    d_out = W_l.shape[0]
    bm = _BM if n_dst % _BM == 0 else 8
    n_blocks = n_dst // bm

    wl_t = W_l.T
    wr_t = W_r.T
    bl = b_l.reshape(1, d_out)

    body = functools.partial(_sage_kernel, bm=bm, n_blocks=n_blocks)

    return pl.pallas_call(
        body,
        grid=(1,),
        in_specs=[
            pl.BlockSpec(memory_space=pl.ANY),
            pl.BlockSpec((n_src, d_in), lambda i: (0, 0)),
            pl.BlockSpec((d_in, d_out), lambda i: (0, 0)),
            pl.BlockSpec((d_in, d_out), lambda i: (0, 0)),
            pl.BlockSpec((1, d_out), lambda i: (0, 0)),
        ],
        out_specs=pl.BlockSpec((n_dst, d_out), lambda i: (0, 0)),
        out_shape=jax.ShapeDtypeStruct((n_dst, d_out), jnp.float32),
        scratch_shapes=[
            pltpu.VMEM((2, bm, n_src), jnp.float32),
            pltpu.SemaphoreType.DMA((2,)),
        ],
    )(adj, x, wl_t, wr_t, bl)


# speedup vs baseline: 1.0693x; 1.0693x over previous
import functools

import jax
import jax.numpy as jnp
from jax.experimental import pallas as pl
from jax.experimental.pallas import tpu as pltpu


def _sage_block_kernel(adj_ref, x_ref, wl_ref, wr_ref, bl_ref, out_ref, *, bm):
    i = pl.program_id(0)
    dot = functools.partial(
        jax.lax.dot_general,
        dimension_numbers=(((1,), (0,)), ((), ())),
        precision=jax.lax.Precision.DEFAULT,
        preferred_element_type=jnp.float32)
    agg = dot(adj_ref[...], x_ref[...])
    out = dot(agg, wl_ref[...])
    x_blk = x_ref[pl.ds(i * bm, bm), :]
    out += dot(x_blk, wr_ref[...])
    out_ref[...] = out + bl_ref[...]


def _pick_bm(n):
    for bm in (400, 200, 100, 80, 40, 8):
        if n % bm == 0:
            return bm
    return n


@jax.jit
def kernel(x, adj, W_l, b_l, W_r):
    n_dst, n_src = adj.shape
    d_in = x.shape[1]
    d_out = W_l.shape[0]
    bm = _pick_bm(n_dst)

    wl_t = W_l.T
    wr_t = W_r.T
    bl = b_l.reshape(1, d_out)

    body = functools.partial(_sage_block_kernel, bm=bm)

    return pl.pallas_call(
        body,
        grid=(n_dst // bm,),
        in_specs=[
            pl.BlockSpec((bm, n_src), lambda i: (i, 0)),
            pl.BlockSpec((n_src, d_in), lambda i: (0, 0)),
            pl.BlockSpec((d_in, d_out), lambda i: (0, 0)),
            pl.BlockSpec((d_in, d_out), lambda i: (0, 0)),
            pl.BlockSpec((1, d_out), lambda i: (0, 0)),
        ],
        out_specs=pl.BlockSpec((bm, d_out), lambda i: (i, 0)),
        out_shape=jax.ShapeDtypeStruct((n_dst, d_out), jnp.float32),
        compiler_params=pltpu.CompilerParams(
            dimension_semantics=("arbitrary",),
        ),
    )(adj, x, wl_t, wr_t, bl)
